# Initial kernel scaffold; baseline (speedup 1.0000x reference)
import jax
import jax.numpy as jnp
from jax.experimental import pallas as pl


def kernel(x, text, w0, scale0, shift0, w1, scale1, shift1, w2, scale2, shift2, w3, scale3, shift3, w4, scale4, shift4, w5, scale5, shift5, w6, scale6, shift6, w_pred, b_pred):
    raise NotImplementedError("write your pallas kernel here")


# speedup vs baseline: 9.1230x; 9.1230x over previous
import functools

import jax
import jax.numpy as jnp
from jax.experimental import pallas as pl
from jax.experimental.pallas import tpu as pltpu

_G = 8

_BODY = (
    (3, 3, 3, 4, 1, (2, 2)),
    (3, 3, 4, 8, 1, (2, 2)),
    (3, 3, 8, 16, 1, None),
    (3, 3, 16, 16, 1, (2, 1)),
    (3, 3, 16, 32, 1, None),
    (3, 3, 32, 32, 1, (2, 1)),
)


def _plan(h, w):
    rows = []
    for (kh, kw, cin, cout, pad, pool) in _BODY:
        ho, wo = h + 2 * pad - kh + 1, w + 2 * pad - kw + 1
        rows.append((kh, kw, cin, cout, pad, pool, h, w, ho, wo))
        h = ho if pool is None else ho // pool[0]
        w = wo if pool is None else wo // pool[1]
    return rows, h, w


def _vgg_ctc_body(plan, x_ref, *refs):
    nl = len(plan)
    lay = refs[:2 * nl]
    w6_ref, b6_ref = refs[2 * nl], refs[2 * nl + 1]
    wp_ref, bp_ref = refs[2 * nl + 2], refs[2 * nl + 3]
    o_ref = refs[2 * nl + 4]
    slab = refs[2 * nl + 5:2 * nl + 5 + nl]
    wps = refs[2 * nl + 5 + nl:]

    cur = x_ref[...]
    wp_i = 0
    for li, (kh, kw, cin, cout, pad, pool, h, w, ho, wo) in enumerate(plan):
        gci, gco = _G * cin, _G * cout
        buf = slab[li]
        hp, wp_w = h + 2 * pad, w + 2 * pad

        buf[:pad, :, :] = jnp.zeros((pad, wp_w, gci), jnp.float32)
        buf[pad + h:, :, :] = jnp.zeros((hp - pad - h, wp_w, gci), jnp.float32)
        buf[pad:pad + h, :pad, :] = jnp.zeros((h, pad, gci), jnp.float32)
        buf[pad:pad + h, pad + w:, :] = jnp.zeros((h, wp_w - pad - w, gci),
                                                  jnp.float32)
        buf[pad:pad + h, pad:pad + w, :] = cur

        acc = jnp.zeros((ho * wo, gco), jnp.float32)
        for dx in range(kw):
            col = buf[:, dx:dx + wo, :].reshape(hp * wo, gci)
            for dy in range(kh):
                acc = acc + jnp.dot(col[dy * wo:(dy + ho) * wo],
                                    lay[2 * li][dy * kw + dx],
                                    preferred_element_type=jnp.float32)

        y = jnp.maximum(acc + lay[2 * li + 1][...], 0.0)

        if pool is None:
            cur = y.reshape(ho, wo, gco)
        else:
            ph, pw = pool
            y4 = y.reshape(ho // ph, ph, wo, gco)
            xh = y4[:, 0]
            for r in range(1, ph):
                xh = jnp.maximum(xh, y4[:, r])
            if pw == 1:
                cur = xh
            else:
                m2 = (ho // ph) * wo
                stage = wps[wp_i]
                wp_i += 1
                stage[...] = xh.reshape(m2, gco)
                pooled = stage[pl.ds(0, m2 // pw, pw), :]
                for r in range(1, pw):
                    pooled = jnp.maximum(pooled,
                                         stage[pl.ds(r, m2 // pw, pw), :])
                cur = pooled.reshape(ho // ph, wo // pw, gco)

    hf, wf_in, gcf = cur.shape
    wo6 = wf_in - 1
    acc = jnp.zeros((wo6, gcf), jnp.float32)
    for dy in range(2):
        for dx in range(2):
            win = cur[dy:dy + 1, dx:dx + wo6, :].reshape(wo6, gcf)
            acc = acc + jnp.dot(win, w6_ref[dy * 2 + dx],
                                preferred_element_type=jnp.float32)
    feat = jnp.maximum(acc + b6_ref[...], 0.0)
    o_ref[...] = (jnp.dot(feat, wp_ref[...], preferred_element_type=jnp.float32)
                  + bp_ref[...])


def _blockdiag(w_taps, g):
    t, ci, co = w_taps.shape
    eye = jnp.eye(g, dtype=w_taps.dtype)
    return (eye[None, :, None, :, None]
            * w_taps[:, None, :, None, :]).reshape(t, g * ci, g * co)


def kernel(x, text, w0, scale0, shift0, w1, scale1, shift1, w2, scale2,
           shift2, w3, scale3, shift3, w4, scale4, shift4, w5, scale5,
           shift5, w6, scale6, shift6, w_pred, b_pred):
    del text
    ws = (w0, w1, w2, w3, w4, w5, w6)
    scales = (scale0, scale1, scale2, scale3, scale4, scale5, scale6)
    shifts = (shift0, shift1, shift2, shift3, shift4, shift5, shift6)

    b, cin0, h, w = x.shape
    nb = b // _G
    plan, hf, wf = _plan(h, w)
    wf -= 1
    nclass = w_pred.shape[1]

    xg = (x.reshape(nb, _G, cin0, h, w)
           .transpose(0, 3, 4, 1, 2)
           .reshape(nb, h, w, _G * cin0)).astype(jnp.float32)

    inputs = [xg]
    in_specs = [pl.BlockSpec((None, h, w, _G * cin0), lambda i: (i, 0, 0, 0))]
    for (kh, kw, ci, co, _p, _q), wl, sc, sh in zip(
            (r[:6] for r in plan), ws[:6], scales[:6], shifts[:6]):
        taps = (wl * sc[None, None, None, :]).reshape(kh * kw, ci, co)
        inputs += [_blockdiag(taps, _G),
                   jnp.tile(sh, _G).reshape(1, _G * co)]
        in_specs += [pl.BlockSpec((kh * kw, _G * ci, _G * co),
                                  lambda i: (0, 0, 0)),
                     pl.BlockSpec((1, _G * co), lambda i: (0, 0))]

    c6 = ws[6].shape[3]
    taps6 = (ws[6] * scales[6][None, None, None, :]).reshape(4, ws[6].shape[2],
                                                             c6)
    inputs += [_blockdiag(taps6, _G), jnp.tile(shifts[6], _G).reshape(1, _G * c6)]
    in_specs += [pl.BlockSpec((4, _G * ws[6].shape[2], _G * c6),
                              lambda i: (0, 0, 0)),
                 pl.BlockSpec((1, _G * c6), lambda i: (0, 0))]

    eye = jnp.eye(_G, dtype=w_pred.dtype)
    wp_big = (eye[:, None, :, None] * w_pred[None, :, None, :]).reshape(
        _G * w_pred.shape[0], _G * nclass)
    inputs += [wp_big, jnp.tile(b_pred, _G).reshape(1, _G * nclass)]
    in_specs += [pl.BlockSpec(wp_big.shape, lambda i: (0, 0)),
                 pl.BlockSpec((1, _G * nclass), lambda i: (0, 0))]

    scratch = [pltpu.VMEM((r[6] + 2 * r[4], r[7] + 2 * r[4], _G * r[2]),
                          jnp.float32) for r in plan]
    scratch += [pltpu.VMEM(((r[8] // r[5][0]) * r[9], _G * r[3]), jnp.float32)
                for r in plan if r[5] is not None and r[5][1] > 1]

    body = functools.partial(_vgg_ctc_body, plan)
    out = pl.pallas_call(
        body,
        out_shape=jax.ShapeDtypeStruct((nb, wf, _G * nclass), jnp.float32),
        grid=(nb,),
        in_specs=in_specs,
        out_specs=pl.BlockSpec((None, wf, _G * nclass), lambda i: (i, 0, 0)),
        scratch_shapes=scratch,
        compiler_params=pltpu.CompilerParams(
            dimension_semantics=("parallel",),
            vmem_limit_bytes=32 * 1024 * 1024),
    )(*inputs)

    return (out.reshape(nb, wf, _G, nclass)
               .transpose(0, 2, 1, 3)
               .reshape(b, wf, nclass))


# speedup vs baseline: 9.7719x; 1.0711x over previous
import functools

import jax
import jax.numpy as jnp
from jax.experimental import pallas as pl
from jax.experimental.pallas import tpu as pltpu

_G = 16

_BODY = (
    (3, 3, 3, 4, 1, (2, 2)),
    (3, 3, 4, 8, 1, (2, 2)),
    (3, 3, 8, 16, 1, None),
    (3, 3, 16, 16, 1, (2, 1)),
    (3, 3, 16, 32, 1, None),
    (3, 3, 32, 32, 1, (2, 1)),
)


def _plan(h, w):
    rows = []
    for (kh, kw, cin, cout, pad, pool) in _BODY:
        ho, wo = h + 2 * pad - kh + 1, w + 2 * pad - kw + 1
        rows.append((kh, kw, cin, cout, pad, pool, h, w, ho, wo))
        h = ho if pool is None else ho // pool[0]
        w = wo if pool is None else wo // pool[1]
    return rows, h, w


def _vgg_ctc_body(plan, x_ref, *refs):
    nl = len(plan)
    lay = refs[:2 * nl]
    w6_ref, b6_ref = refs[2 * nl], refs[2 * nl + 1]
    wp_ref, bp_ref = refs[2 * nl + 2], refs[2 * nl + 3]
    o_ref = refs[2 * nl + 4]
    slab = refs[2 * nl + 5:2 * nl + 5 + nl]
    wps = refs[2 * nl + 5 + nl:]

    cur = x_ref[...]
    wp_i = 0
    for li, (kh, kw, cin, cout, pad, pool, h, w, ho, wo) in enumerate(plan):
        gci, gco = _G * cin, _G * cout
        buf = slab[li]
        hp, wp_w = h + 2 * pad, w + 2 * pad

        buf[:pad, :, :] = jnp.zeros((pad, wp_w, gci), jnp.float32)
        buf[pad + h:, :, :] = jnp.zeros((hp - pad - h, wp_w, gci), jnp.float32)
        buf[pad:pad + h, :pad, :] = jnp.zeros((h, pad, gci), jnp.float32)
        buf[pad:pad + h, pad + w:, :] = jnp.zeros((h, wp_w - pad - w, gci),
                                                  jnp.float32)
        buf[pad:pad + h, pad:pad + w, :] = cur

        acc = jnp.zeros((ho * wo, gco), jnp.float32)
        for dx in range(kw):
            col = buf[:, dx:dx + wo, :].reshape(hp * wo, gci)
            for dy in range(kh):
                acc = acc + jnp.dot(col[dy * wo:(dy + ho) * wo],
                                    lay[2 * li][dy * kw + dx],
                                    preferred_element_type=jnp.float32)

        y = jnp.maximum(acc + lay[2 * li + 1][...], 0.0)

        if pool is None:
            cur = y.reshape(ho, wo, gco)
        else:
            ph, pw = pool
            y4 = y.reshape(ho // ph, ph, wo, gco)
            xh = y4[:, 0]
            for r in range(1, ph):
                xh = jnp.maximum(xh, y4[:, r])
            if pw == 1:
                cur = xh
            else:
                m2 = (ho // ph) * wo
                stage = wps[wp_i]
                wp_i += 1
                stage[...] = xh.reshape(m2, gco)
                pooled = stage[pl.ds(0, m2 // pw, pw), :]
                for r in range(1, pw):
                    pooled = jnp.maximum(pooled,
                                         stage[pl.ds(r, m2 // pw, pw), :])
                cur = pooled.reshape(ho // ph, wo // pw, gco)

    hf, wf_in, gcf = cur.shape
    wo6 = wf_in - 1
    acc = jnp.zeros((wo6, gcf), jnp.float32)
    for dy in range(2):
        for dx in range(2):
            win = cur[dy:dy + 1, dx:dx + wo6, :].reshape(wo6, gcf)
            acc = acc + jnp.dot(win, w6_ref[dy * 2 + dx],
                                preferred_element_type=jnp.float32)
    feat = jnp.maximum(acc + b6_ref[...], 0.0)
    o_ref[...] = (jnp.dot(feat, wp_ref[...], preferred_element_type=jnp.float32)
                  + bp_ref[...])


def _blockdiag(w_taps, g):
    t, ci, co = w_taps.shape
    eye = jnp.eye(g, dtype=w_taps.dtype)
    return (eye[None, :, None, :, None]
            * w_taps[:, None, :, None, :]).reshape(t, g * ci, g * co)


def kernel(x, text, w0, scale0, shift0, w1, scale1, shift1, w2, scale2,
           shift2, w3, scale3, shift3, w4, scale4, shift4, w5, scale5,
           shift5, w6, scale6, shift6, w_pred, b_pred):
    del text
    ws = (w0, w1, w2, w3, w4, w5, w6)
    scales = (scale0, scale1, scale2, scale3, scale4, scale5, scale6)
    shifts = (shift0, shift1, shift2, shift3, shift4, shift5, shift6)

    b, cin0, h, w = x.shape
    nb = b // _G
    plan, hf, wf = _plan(h, w)
    wf -= 1
    nclass = w_pred.shape[1]

    xg = (x.reshape(nb, _G, cin0, h, w)
           .transpose(0, 3, 4, 1, 2)
           .reshape(nb, h, w, _G * cin0)).astype(jnp.float32)

    inputs = [xg]
    in_specs = [pl.BlockSpec((None, h, w, _G * cin0), lambda i: (i, 0, 0, 0))]
    for (kh, kw, ci, co, _p, _q), wl, sc, sh in zip(
            (r[:6] for r in plan), ws[:6], scales[:6], shifts[:6]):
        taps = (wl * sc[None, None, None, :]).reshape(kh * kw, ci, co)
        inputs += [_blockdiag(taps, _G),
                   jnp.tile(sh, _G).reshape(1, _G * co)]
        in_specs += [pl.BlockSpec((kh * kw, _G * ci, _G * co),
                                  lambda i: (0, 0, 0)),
                     pl.BlockSpec((1, _G * co), lambda i: (0, 0))]

    c6 = ws[6].shape[3]
    taps6 = (ws[6] * scales[6][None, None, None, :]).reshape(4, ws[6].shape[2],
                                                             c6)
    inputs += [_blockdiag(taps6, _G), jnp.tile(shifts[6], _G).reshape(1, _G * c6)]
    in_specs += [pl.BlockSpec((4, _G * ws[6].shape[2], _G * c6),
                              lambda i: (0, 0, 0)),
                 pl.BlockSpec((1, _G * c6), lambda i: (0, 0))]

    eye = jnp.eye(_G, dtype=w_pred.dtype)
    wp_big = (eye[:, None, :, None] * w_pred[None, :, None, :]).reshape(
        _G * w_pred.shape[0], _G * nclass)
    inputs += [wp_big, jnp.tile(b_pred, _G).reshape(1, _G * nclass)]
    in_specs += [pl.BlockSpec(wp_big.shape, lambda i: (0, 0)),
                 pl.BlockSpec((1, _G * nclass), lambda i: (0, 0))]

    scratch = [pltpu.VMEM((r[6] + 2 * r[4], r[7] + 2 * r[4], _G * r[2]),
                          jnp.float32) for r in plan]
    scratch += [pltpu.VMEM(((r[8] // r[5][0]) * r[9], _G * r[3]), jnp.float32)
                for r in plan if r[5] is not None and r[5][1] > 1]

    body = functools.partial(_vgg_ctc_body, plan)
    out = pl.pallas_call(
        body,
        out_shape=jax.ShapeDtypeStruct((nb, wf, _G * nclass), jnp.float32),
        grid=(nb,),
        in_specs=in_specs,
        out_specs=pl.BlockSpec((None, wf, _G * nclass), lambda i: (i, 0, 0)),
        scratch_shapes=scratch,
        compiler_params=pltpu.CompilerParams(
            dimension_semantics=("parallel",),
            vmem_limit_bytes=32 * 1024 * 1024),
    )(*inputs)

    return (out.reshape(nb, wf, _G, nclass)
               .transpose(0, 2, 1, 3)
               .reshape(b, wf, nclass))
